# Initial kernel scaffold; baseline (speedup 1.0000x reference)
import functools

import jax
import jax.numpy as jnp
from jax import lax
from jax.experimental import pallas as pl
from jax.experimental.pallas import tpu as pltpu
from jax.experimental.pallas import tpu_sc as plsc

N = 10000
D = 128
E = 320000
NC = 2
NS = 16
NW = NC * NS
EPW = E // NW
CH = 80
NCHUNK = EPW // CH
RPT = N // NS
ZR = 125


def _sc_aggregate(h, src, dst):
    mesh = plsc.VectorSubcoreMesh(core_axis_name="c", subcore_axis_name="s")

    @functools.partial(
        pl.kernel,
        mesh=mesh,
        out_type=jax.ShapeDtypeStruct((NC * N, D), jnp.float32),
        scratch_types=[
            pltpu.VMEM((CH,), jnp.int32),
            pltpu.VMEM((CH,), jnp.int32),
            pltpu.VMEM((CH, D), jnp.float32),
            pltpu.VMEM((ZR, D), jnp.float32),
            pltpu.VMEM_SHARED((N, D), jnp.float32),
            pltpu.SemaphoreType.DMA,
        ],
    )
    def agg_kernel(h_hbm, src_hbm, dst_hbm, out_hbm, sidx, didx, rows, zbuf, acc, sem):
        cid = lax.axis_index("c")
        sid = lax.axis_index("s")
        wid = cid * NS + sid

        z = jnp.zeros((16,), jnp.float32)

        def zstore(i, _):
            r = i // (D // 16)
            k = i % (D // 16)
            zbuf[r, pl.ds(k * 16, 16)] = z
            return 0

        lax.fori_loop(0, ZR * (D // 16), zstore, 0)

        def zcopy(j, _):
            pltpu.sync_copy(zbuf, acc.at[pl.ds(sid * RPT + j * ZR, ZR)])
            return 0

        lax.fori_loop(0, RPT // ZR, zcopy, 0)
        plsc.subcore_barrier()

        base0 = wid * EPW

        def body(i, _):
            base = base0 + i * CH
            pltpu.sync_copy(src_hbm.at[pl.ds(base, CH)], sidx)
            pltpu.sync_copy(dst_hbm.at[pl.ds(base, CH)], didx)
            pltpu.async_copy(h_hbm.at[sidx], rows, sem).wait()
            pltpu.sync_copy(rows, acc.at[didx], add=True)
            return 0

        lax.fori_loop(0, NCHUNK, body, 0)
        plsc.subcore_barrier()

        pltpu.sync_copy(
            acc.at[pl.ds(sid * RPT, RPT)],
            out_hbm.at[pl.ds(cid * N + sid * RPT, RPT)],
        )

    return agg_kernel(h, src, dst)


_BR = 1000


def _tc_linear(x, W, b):
    def body(x_ref, w_ref, b_ref, o_ref):
        o_ref[...] = (
            jnp.dot(x_ref[...], w_ref[...], preferred_element_type=jnp.float32)
            + b_ref[...]
        )

    return pl.pallas_call(
        body,
        grid=(N // _BR,),
        in_specs=[
            pl.BlockSpec((_BR, D), lambda i: (i, 0)),
            pl.BlockSpec((D, D), lambda i: (0, 0)),
            pl.BlockSpec((1, D), lambda i: (0, 0)),
        ],
        out_specs=pl.BlockSpec((_BR, D), lambda i: (i, 0)),
        out_shape=jax.ShapeDtypeStruct((N, D), jnp.float32),
    )(x, W, b.reshape(1, D))


def _tc_add_relu_linear(p, W, b):
    def body(p0_ref, p1_ref, w_ref, b_ref, o_ref):
        hloc = jnp.maximum(p0_ref[...] + p1_ref[...], 0.0)
        o_ref[...] = (
            jnp.dot(hloc, w_ref[...], preferred_element_type=jnp.float32)
            + b_ref[...]
        )

    nb = N // _BR
    return pl.pallas_call(
        body,
        grid=(nb,),
        in_specs=[
            pl.BlockSpec((_BR, D), lambda i: (i, 0)),
            pl.BlockSpec((_BR, D), lambda i: (i + nb, 0)),
            pl.BlockSpec((D, D), lambda i: (0, 0)),
            pl.BlockSpec((1, D), lambda i: (0, 0)),
        ],
        out_specs=pl.BlockSpec((_BR, D), lambda i: (i, 0)),
        out_shape=jax.ShapeDtypeStruct((N, D), jnp.float32),
    )(p, p, W, b.reshape(1, D))


def _tc_add_relu_norm(p):
    def body(p0_ref, p1_ref, o_ref):
        y = jnp.maximum(p0_ref[...] + p1_ref[...], 0.0)
        nrm = jnp.sqrt(jnp.sum(y * y, axis=-1, keepdims=True))
        o_ref[...] = y / jnp.maximum(nrm, 1e-12)

    nb = N // _BR
    return pl.pallas_call(
        body,
        grid=(nb,),
        in_specs=[
            pl.BlockSpec((_BR, D), lambda i: (i, 0)),
            pl.BlockSpec((_BR, D), lambda i: (i + nb, 0)),
        ],
        out_specs=pl.BlockSpec((_BR, D), lambda i: (i, 0)),
        out_shape=jax.ShapeDtypeStruct((N, D), jnp.float32),
    )(p, p)


def kernel(x, edge_index, W0, b0, W1, b1):
    src = edge_index[0]
    dst = edge_index[1]
    h1 = _tc_linear(x, W0, b0)
    p1 = _sc_aggregate(h1, src, dst)
    h2 = _tc_add_relu_linear(p1, W1, b1)
    p2 = _sc_aggregate(h2, src, dst)
    return _tc_add_relu_norm(p2)


# speedup vs baseline: 5.1365x; 5.1365x over previous
import functools

import jax
import jax.numpy as jnp
from jax import lax
from jax.experimental import pallas as pl
from jax.experimental.pallas import tpu as pltpu
from jax.experimental.pallas import tpu_sc as plsc

N = 10000
D = 128
E = 320000
NC = 2
NS = 16
NW = NC * NS
EPW = E // NW
CH = 80
NCHUNK = EPW // CH
NP = 10240
RPT = NP // NS


def _sc_aggregate(h, src, dst):
    mesh = plsc.VectorSubcoreMesh(core_axis_name="c", subcore_axis_name="s")

    @functools.partial(
        pl.kernel,
        mesh=mesh,
        out_type=[
            jax.ShapeDtypeStruct((NP, D), jnp.float32),
            jax.ShapeDtypeStruct((NP, D), jnp.float32),
        ],
        scratch_types=[
            pltpu.VMEM((CH,), jnp.int32),
            pltpu.VMEM((CH,), jnp.int32),
            pltpu.VMEM((CH, D), jnp.float32),
            pltpu.VMEM_SHARED((NP, D), jnp.float32),
            pltpu.SemaphoreType.DMA,
        ],
    )
    def agg_kernel(h_hbm, src_hbm, dst_hbm, out0, out1, sidx, didx, rows, acc, sem):
        cid = lax.axis_index("c")
        sid = lax.axis_index("s")
        wid = cid * NS + sid

        z = jnp.zeros((16,), jnp.float32)

        def zstore(i, _):
            r = i // (D // 16)
            k = i % (D // 16)
            rows[r, pl.ds(k * 16, 16)] = z
            return 0

        lax.fori_loop(0, CH * (D // 16), zstore, 0)

        def zcopy(j, _):
            pltpu.sync_copy(rows, acc.at[pl.ds(sid * RPT + j * CH, CH)])
            return 0

        lax.fori_loop(0, RPT // CH, zcopy, 0)
        plsc.subcore_barrier()

        base0 = wid * EPW

        def body(i, _):
            base = base0 + i * CH
            pltpu.sync_copy(src_hbm.at[pl.ds(base, CH)], sidx)
            pltpu.sync_copy(dst_hbm.at[pl.ds(base, CH)], didx)
            pltpu.async_copy(h_hbm.at[sidx], rows, sem).wait()
            pltpu.sync_copy(rows, acc.at[didx], add=True)
            return 0

        lax.fori_loop(0, NCHUNK, body, 0)
        plsc.subcore_barrier()

        @pl.when(cid == 0)
        def _():
            pltpu.sync_copy(acc.at[pl.ds(sid * RPT, RPT)],
                            out0.at[pl.ds(sid * RPT, RPT)])

        @pl.when(cid == 1)
        def _():
            pltpu.sync_copy(acc.at[pl.ds(sid * RPT, RPT)],
                            out1.at[pl.ds(sid * RPT, RPT)])

    return agg_kernel(h, src, dst)


_BR = 1000


def _tc_linear(x, W, b):
    def body(x_ref, w_ref, b_ref, o_ref):
        o_ref[...] = (
            jnp.dot(x_ref[...], w_ref[...], preferred_element_type=jnp.float32)
            + b_ref[...]
        )

    return pl.pallas_call(
        body,
        grid=(N // _BR,),
        in_specs=[
            pl.BlockSpec((_BR, D), lambda i: (i, 0)),
            pl.BlockSpec((D, D), lambda i: (0, 0)),
            pl.BlockSpec((1, D), lambda i: (0, 0)),
        ],
        out_specs=pl.BlockSpec((_BR, D), lambda i: (i, 0)),
        out_shape=jax.ShapeDtypeStruct((N, D), jnp.float32),
    )(x, W, b.reshape(1, D))


def _tc_add_relu_linear(p0, p1, W, b):
    def body(p0_ref, p1_ref, w_ref, b_ref, o_ref):
        hloc = jnp.maximum(p0_ref[...] + p1_ref[...], 0.0)
        o_ref[...] = (
            jnp.dot(hloc, w_ref[...], preferred_element_type=jnp.float32)
            + b_ref[...]
        )

    return pl.pallas_call(
        body,
        grid=(N // _BR,),
        in_specs=[
            pl.BlockSpec((_BR, D), lambda i: (i, 0)),
            pl.BlockSpec((_BR, D), lambda i: (i, 0)),
            pl.BlockSpec((D, D), lambda i: (0, 0)),
            pl.BlockSpec((1, D), lambda i: (0, 0)),
        ],
        out_specs=pl.BlockSpec((_BR, D), lambda i: (i, 0)),
        out_shape=jax.ShapeDtypeStruct((N, D), jnp.float32),
    )(p0, p1, W, b.reshape(1, D))


def _tc_add_relu_norm(p0, p1):
    def body(p0_ref, p1_ref, o_ref):
        y = jnp.maximum(p0_ref[...] + p1_ref[...], 0.0)
        nrm = jnp.sqrt(jnp.sum(y * y, axis=-1, keepdims=True))
        o_ref[...] = y / jnp.maximum(nrm, 1e-12)

    return pl.pallas_call(
        body,
        grid=(N // _BR,),
        in_specs=[
            pl.BlockSpec((_BR, D), lambda i: (i, 0)),
            pl.BlockSpec((_BR, D), lambda i: (i, 0)),
        ],
        out_specs=pl.BlockSpec((_BR, D), lambda i: (i, 0)),
        out_shape=jax.ShapeDtypeStruct((N, D), jnp.float32),
    )(p0, p1)


def kernel(x, edge_index, W0, b0, W1, b1):
    src = edge_index[0]
    dst = edge_index[1]
    h1 = _tc_linear(x, W0, b0)
    a0, a1 = _sc_aggregate(h1, src, dst)
    h2 = _tc_add_relu_linear(a0, a1, W1, b1)
    c0, c1 = _sc_aggregate(h2, src, dst)
    return _tc_add_relu_norm(c0, c1)
